# baseline (device time: 153440 ns/iter reference)
import jax
import jax.numpy as jnp
from jax import lax
from jax.experimental import pallas as pl
from jax.experimental.pallas import tpu as pltpu

N_Z = 4


def kernel(partial, resid, gamma):
    m, d = resid.shape
    x2d = partial.reshape(m, d)
    gamma2d = gamma.reshape(1, d)

    def body(x_ref, resid_ref, gamma_ref, out_ref, comm_ref, send_sems, recv_sems):
        my_x = lax.axis_index("x")
        my_y = lax.axis_index("y")
        my_z = lax.axis_index("z")
        left = (my_z - 1) % N_Z
        right = (my_z + 1) % N_Z

        barrier_sem = pltpu.get_barrier_semaphore()
        for nbr in [left, right]:
            pl.semaphore_signal(
                barrier_sem,
                inc=1,
                device_id=(my_x, my_y, nbr),
                device_id_type=pl.DeviceIdType.MESH,
            )
        pl.semaphore_wait(barrier_sem, 2)

        out_ref[...] = x_ref[...] + resid_ref[...]
        comm_ref[0] = x_ref[...]

        for h in range(N_Z - 1):
            rdma = pltpu.make_async_remote_copy(
                src_ref=comm_ref.at[h],
                dst_ref=comm_ref.at[h + 1],
                send_sem=send_sems.at[h],
                recv_sem=recv_sems.at[h],
                device_id=(my_x, my_y, right),
                device_id_type=pl.DeviceIdType.MESH,
            )
            rdma.start()
            rdma.wait()
            out_ref[...] += comm_ref[h + 1]

        y = out_ref[...]
        rms = jnp.sqrt(jnp.mean(y * y, axis=-1, keepdims=True) + 1e-6)
        out_ref[...] = y / rms * gamma_ref[...]

    return pl.pallas_call(
        body,
        out_shape=jax.ShapeDtypeStruct((m, d), jnp.float32),
        in_specs=[
            pl.BlockSpec(memory_space=pltpu.VMEM),
            pl.BlockSpec(memory_space=pltpu.VMEM),
            pl.BlockSpec(memory_space=pltpu.VMEM),
        ],
        out_specs=pl.BlockSpec(memory_space=pltpu.VMEM),
        scratch_shapes=[
            pltpu.VMEM((N_Z, m, d), jnp.float32),
            pltpu.SemaphoreType.DMA((N_Z - 1,)),
            pltpu.SemaphoreType.DMA((N_Z - 1,)),
        ],
        compiler_params=pltpu.CompilerParams(collective_id=0),
    )(x2d, resid, gamma2d)


# device time: 59309 ns/iter; 2.5871x vs baseline; 2.5871x over previous
import jax
import jax.numpy as jnp
from jax import lax
from jax.experimental import pallas as pl
from jax.experimental.pallas import tpu as pltpu

N_Z = 4
BLK = 64
GRP = 256

_MESH = pl.DeviceIdType.MESH


def kernel(partial, resid, gamma):
    m, d = resid.shape
    x2d = partial.reshape(m, d)
    gamma2d = gamma.reshape(1, d)

    def body(x_ref, resid_ref, gamma_ref, out_ref,
             rs_buf, rs_recv, rs_send,
             agz_recv, agz_send,
             agxy_recv, agxy_send):
        my_x = lax.axis_index("x")
        my_y = lax.axis_index("y")
        my_z = lax.axis_index("z")
        p = 2 * my_x + my_y
        grp0 = GRP * p
        myrow = grp0 + BLK * my_z

        def copy(src, dst, send_sem, recv_sem, dev):
            return pltpu.make_async_remote_copy(
                src_ref=src, dst_ref=dst, send_sem=send_sem,
                recv_sem=recv_sem, device_id=dev, device_id_type=_MESH)

        barrier_sem = pltpu.get_barrier_semaphore()
        for dz in range(1, N_Z):
            pl.semaphore_signal(barrier_sem, inc=1,
                                device_id=(my_x, my_y, (my_z + dz) % N_Z),
                                device_id_type=_MESH)
        pl.semaphore_signal(barrier_sem, inc=1,
                            device_id=(1 - my_x, my_y, my_z),
                            device_id_type=_MESH)
        pl.semaphore_signal(barrier_sem, inc=1,
                            device_id=(my_x, 1 - my_y, my_z),
                            device_id_type=_MESH)
        pl.semaphore_wait(barrier_sem, 5)

        started = []

        for dz in range(1, N_Z):
            tz = (my_z + dz) % N_Z
            rdma = copy(x_ref.at[pl.ds(grp0 + BLK * tz, BLK), :],
                        rs_buf.at[pl.ds(my_z * BLK, BLK), :],
                        rs_send.at[dz - 1], rs_recv.at[my_z],
                        (my_x, my_y, tz))
            rdma.start()
            started.append(rdma)
        rs_buf[pl.ds(my_z * BLK, BLK), :] = x_ref[pl.ds(myrow, BLK), :]
        for dz in range(1, N_Z):
            sz = (my_z + dz) % N_Z
            copy(rs_buf.at[pl.ds(sz * BLK, BLK), :],
                 rs_buf.at[pl.ds(sz * BLK, BLK), :],
                 rs_send.at[dz - 1], rs_recv.at[sz],
                 (my_x, my_y, sz)).wait_recv()

        y = resid_ref[pl.ds(myrow, BLK), :]
        for s in range(N_Z):
            y = y + rs_buf[s * BLK:(s + 1) * BLK, :]
        rms = jnp.sqrt(jnp.mean(y * y, axis=-1, keepdims=True) + 1e-6)
        out_ref[pl.ds(myrow, BLK), :] = y / rms * gamma_ref[...]

        for dz in range(1, N_Z):
            tz = (my_z + dz) % N_Z
            rdma = copy(out_ref.at[pl.ds(myrow, BLK), :],
                        out_ref.at[pl.ds(myrow, BLK), :],
                        agz_send.at[dz - 1], agz_recv.at[my_z],
                        (my_x, my_y, tz))
            rdma.start()
            started.append(rdma)
        for dz in range(1, N_Z):
            sz = (my_z + dz) % N_Z
            copy(out_ref.at[pl.ds(grp0 + BLK * sz, BLK), :],
                 out_ref.at[pl.ds(grp0 + BLK * sz, BLK), :],
                 agz_send.at[dz - 1], agz_recv.at[sz],
                 (my_x, my_y, sz)).wait_recv()

        p_x = 2 * (1 - my_x) + my_y
        p_y = 2 * my_x + (1 - my_y)
        p_dg = 2 * (1 - my_x) + (1 - my_y)
        r_to_x = copy(out_ref.at[pl.ds(grp0, GRP), :],
                      out_ref.at[pl.ds(grp0, GRP), :],
                      agxy_send.at[0], agxy_recv.at[0],
                      (1 - my_x, my_y, my_z))
        r_to_y = copy(out_ref.at[pl.ds(grp0, GRP), :],
                      out_ref.at[pl.ds(grp0, GRP), :],
                      agxy_send.at[1], agxy_recv.at[1],
                      (my_x, 1 - my_y, my_z))
        r_to_x.start()
        r_to_y.start()
        started += [r_to_x, r_to_y]
        copy(out_ref.at[pl.ds(GRP * p_x, GRP), :],
             out_ref.at[pl.ds(GRP * p_x, GRP), :],
             agxy_send.at[0], agxy_recv.at[0],
             (1 - my_x, my_y, my_z)).wait_recv()
        copy(out_ref.at[pl.ds(GRP * p_y, GRP), :],
             out_ref.at[pl.ds(GRP * p_y, GRP), :],
             agxy_send.at[1], agxy_recv.at[1],
             (my_x, 1 - my_y, my_z)).wait_recv()

        r_fwd = copy(out_ref.at[pl.ds(GRP * p_x, GRP), :],
                     out_ref.at[pl.ds(GRP * p_x, GRP), :],
                     agxy_send.at[2], agxy_recv.at[2],
                     (my_x, 1 - my_y, my_z))
        r_fwd.start()
        started.append(r_fwd)
        copy(out_ref.at[pl.ds(GRP * p_dg, GRP), :],
             out_ref.at[pl.ds(GRP * p_dg, GRP), :],
             agxy_send.at[2], agxy_recv.at[2],
             (my_x, 1 - my_y, my_z)).wait_recv()

        for rdma in started:
            rdma.wait_send()

    return pl.pallas_call(
        body,
        out_shape=jax.ShapeDtypeStruct((m, d), jnp.float32),
        in_specs=[
            pl.BlockSpec(memory_space=pltpu.VMEM),
            pl.BlockSpec(memory_space=pltpu.VMEM),
            pl.BlockSpec(memory_space=pltpu.VMEM),
        ],
        out_specs=pl.BlockSpec(memory_space=pltpu.VMEM),
        scratch_shapes=[
            pltpu.VMEM((N_Z * BLK, d), jnp.float32),
            pltpu.SemaphoreType.DMA((N_Z,)),
            pltpu.SemaphoreType.DMA((N_Z - 1,)),
            pltpu.SemaphoreType.DMA((N_Z,)),
            pltpu.SemaphoreType.DMA((N_Z - 1,)),
            pltpu.SemaphoreType.DMA((3,)),
            pltpu.SemaphoreType.DMA((3,)),
        ],
        compiler_params=pltpu.CompilerParams(collective_id=0),
    )(x2d, resid, gamma2d)


# device time: 49035 ns/iter; 3.1292x vs baseline; 1.2095x over previous
import jax
import jax.numpy as jnp
from jax import lax
from jax.experimental import pallas as pl
from jax.experimental.pallas import tpu as pltpu

N_Z = 4
BLK = 64
GRP = 256

_MESH = pl.DeviceIdType.MESH


def kernel(partial, resid, gamma):
    m, d = resid.shape
    x2d = partial.reshape(m, d)
    gamma2d = gamma.reshape(1, d)

    def body(x_ref, resid_ref, gamma_ref, out_ref,
             rs_buf, rs_recv, rs_send,
             agz_recv, agz_send,
             agx_recv, agy_recv, agd_recv,
             sendx_sems, sendy_sems):
        my_x = lax.axis_index("x")
        my_y = lax.axis_index("y")
        my_z = lax.axis_index("z")
        p = 2 * my_x + my_y
        grp0 = GRP * p
        myrow = grp0 + BLK * my_z
        xnbr = (1 - my_x, my_y, my_z)
        ynbr = (my_x, 1 - my_y, my_z)
        p_x = 2 * (1 - my_x) + my_y
        p_y = 2 * my_x + (1 - my_y)
        p_dg = 2 * (1 - my_x) + (1 - my_y)

        def copy(src, dst, send_sem, recv_sem, dev):
            return pltpu.make_async_remote_copy(
                src_ref=src, dst_ref=dst, send_sem=send_sem,
                recv_sem=recv_sem, device_id=dev, device_id_type=_MESH)

        barrier_sem = pltpu.get_barrier_semaphore()
        for dz in range(1, N_Z):
            pl.semaphore_signal(barrier_sem, inc=1,
                                device_id=(my_x, my_y, (my_z + dz) % N_Z),
                                device_id_type=_MESH)
        pl.semaphore_signal(barrier_sem, inc=1, device_id=xnbr,
                            device_id_type=_MESH)
        pl.semaphore_signal(barrier_sem, inc=1, device_id=ynbr,
                            device_id_type=_MESH)
        pl.semaphore_wait(barrier_sem, 5)

        started = []
        nx = iter(range(8))
        ny = iter(range(8))

        def send_x(src, dst, recv_sem):
            r = copy(src, dst, sendx_sems.at[next(nx)], recv_sem, xnbr)
            r.start()
            started.append(r)

        def send_y(src, dst, recv_sem):
            r = copy(src, dst, sendy_sems.at[next(ny)], recv_sem, ynbr)
            r.start()
            started.append(r)

        for dz in range(1, N_Z):
            tz = (my_z + dz) % N_Z
            rdma = copy(x_ref.at[pl.ds(grp0 + BLK * tz, BLK), :],
                        rs_buf.at[pl.ds(my_z * BLK, BLK), :],
                        rs_send.at[dz - 1], rs_recv.at[my_z],
                        (my_x, my_y, tz))
            rdma.start()
            started.append(rdma)
        y = x_ref[pl.ds(myrow, BLK), :] + resid_ref[pl.ds(myrow, BLK), :]
        for dz in range(1, N_Z):
            sz = (my_z + dz) % N_Z
            copy(rs_buf.at[pl.ds(sz * BLK, BLK), :],
                 rs_buf.at[pl.ds(sz * BLK, BLK), :],
                 rs_send.at[dz - 1], rs_recv.at[sz],
                 (my_x, my_y, sz)).wait_recv()
            y = y + rs_buf[pl.ds(sz * BLK, BLK), :]

        rms = jnp.sqrt(jnp.mean(y * y, axis=-1, keepdims=True) + 1e-6)
        out_ref[pl.ds(myrow, BLK), :] = y / rms * gamma_ref[...]

        my_blk = out_ref.at[pl.ds(myrow, BLK), :]
        for dz in range(1, N_Z):
            tz = (my_z + dz) % N_Z
            rdma = copy(my_blk, my_blk, agz_send.at[dz - 1],
                        agz_recv.at[my_z], (my_x, my_y, tz))
            rdma.start()
            started.append(rdma)
        send_x(my_blk, my_blk, agx_recv.at[my_z])
        send_y(my_blk, my_blk, agy_recv.at[my_z])

        for dz in range(1, N_Z):
            sz = (my_z + dz) % N_Z
            blk = out_ref.at[pl.ds(grp0 + BLK * sz, BLK), :]
            copy(blk, blk, agz_send.at[dz - 1], agz_recv.at[sz],
                 (my_x, my_y, sz)).wait_recv()
            send_x(blk, blk, agx_recv.at[sz])
            send_y(blk, blk, agy_recv.at[sz])

        for z in range(N_Z):
            bx = out_ref.at[pl.ds(GRP * p_x + BLK * z, BLK), :]
            copy(bx, bx, sendx_sems.at[0], agx_recv.at[z], xnbr).wait_recv()
            if z % 2 == 1:
                send_y(bx, bx, agd_recv.at[z])
            by = out_ref.at[pl.ds(GRP * p_y + BLK * z, BLK), :]
            copy(by, by, sendy_sems.at[0], agy_recv.at[z], ynbr).wait_recv()
            if z % 2 == 0:
                send_x(by, by, agd_recv.at[z])

        for z in range(N_Z):
            bd = out_ref.at[pl.ds(GRP * p_dg + BLK * z, BLK), :]
            src = xnbr if z % 2 == 0 else ynbr
            copy(bd, bd, sendx_sems.at[0], agd_recv.at[z], src).wait_recv()

        for rdma in started:
            rdma.wait_send()

    return pl.pallas_call(
        body,
        out_shape=jax.ShapeDtypeStruct((m, d), jnp.float32),
        in_specs=[
            pl.BlockSpec(memory_space=pltpu.VMEM),
            pl.BlockSpec(memory_space=pltpu.VMEM),
            pl.BlockSpec(memory_space=pltpu.VMEM),
        ],
        out_specs=pl.BlockSpec(memory_space=pltpu.VMEM),
        scratch_shapes=[
            pltpu.VMEM((N_Z * BLK, d), jnp.float32),
            pltpu.SemaphoreType.DMA((N_Z,)),
            pltpu.SemaphoreType.DMA((N_Z - 1,)),
            pltpu.SemaphoreType.DMA((N_Z,)),
            pltpu.SemaphoreType.DMA((N_Z - 1,)),
            pltpu.SemaphoreType.DMA((N_Z,)),
            pltpu.SemaphoreType.DMA((N_Z,)),
            pltpu.SemaphoreType.DMA((N_Z,)),
            pltpu.SemaphoreType.DMA((8,)),
            pltpu.SemaphoreType.DMA((8,)),
        ],
        compiler_params=pltpu.CompilerParams(collective_id=0),
    )(x2d, resid, gamma2d)


# device time: 45180 ns/iter; 3.3962x vs baseline; 1.0853x over previous
import jax
import jax.numpy as jnp
from jax import lax
from jax.experimental import pallas as pl
from jax.experimental.pallas import tpu as pltpu

N_Z = 4
BLK = 64
HB = 32
GRP = 256

_MESH = pl.DeviceIdType.MESH


def kernel(partial, resid, gamma):
    m, d = resid.shape
    x2d = partial.reshape(m, d)
    gamma2d = gamma.reshape(1, d)

    def body(x_ref, resid_ref, gamma_ref, out_ref,
             rs_buf, rs_recv, rs_send,
             agz_recv, agz_send,
             agx_recv, agy_recv, agd_recv,
             sendx_sems, sendy_sems):
        my_x = lax.axis_index("x")
        my_y = lax.axis_index("y")
        my_z = lax.axis_index("z")
        p = 2 * my_x + my_y
        grp0 = GRP * p
        myrow = grp0 + BLK * my_z
        xnbr = (1 - my_x, my_y, my_z)
        ynbr = (my_x, 1 - my_y, my_z)
        p_x = 2 * (1 - my_x) + my_y
        p_y = 2 * my_x + (1 - my_y)
        p_dg = 2 * (1 - my_x) + (1 - my_y)

        def copy(src, dst, send_sem, recv_sem, dev):
            return pltpu.make_async_remote_copy(
                src_ref=src, dst_ref=dst, send_sem=send_sem,
                recv_sem=recv_sem, device_id=dev, device_id_type=_MESH)

        barrier_sem = pltpu.get_barrier_semaphore()
        for dz in range(1, N_Z):
            pl.semaphore_signal(barrier_sem, inc=1,
                                device_id=(my_x, my_y, (my_z + dz) % N_Z),
                                device_id_type=_MESH)
        pl.semaphore_signal(barrier_sem, inc=1, device_id=xnbr,
                            device_id_type=_MESH)
        pl.semaphore_signal(barrier_sem, inc=1, device_id=ynbr,
                            device_id_type=_MESH)
        pl.semaphore_wait(barrier_sem, 5)

        started = []
        nx = iter(range(12))
        ny = iter(range(12))

        def send_x(ref, recv_sem):
            r = copy(ref, ref, sendx_sems.at[next(nx)], recv_sem, xnbr)
            r.start()
            started.append(r)

        def send_y(ref, recv_sem):
            r = copy(ref, ref, sendy_sems.at[next(ny)], recv_sem, ynbr)
            r.start()
            started.append(r)

        for h in range(2):
            for dz in range(1, N_Z):
                tz = (my_z + dz) % N_Z
                rdma = copy(x_ref.at[pl.ds(grp0 + BLK * tz + HB * h, HB), :],
                            rs_buf.at[pl.ds(my_z * BLK + HB * h, HB), :],
                            rs_send.at[dz - 1, h], rs_recv.at[my_z, h],
                            (my_x, my_y, tz))
                rdma.start()
                started.append(rdma)

        for h in range(2):
            row_h = myrow + HB * h
            y = (x_ref[pl.ds(row_h, HB), :]
                 + resid_ref[pl.ds(row_h, HB), :])
            for dz in range(1, N_Z):
                sz = (my_z + dz) % N_Z
                copy(rs_buf.at[pl.ds(sz * BLK + HB * h, HB), :],
                     rs_buf.at[pl.ds(sz * BLK + HB * h, HB), :],
                     rs_send.at[dz - 1, h], rs_recv.at[sz, h],
                     (my_x, my_y, sz)).wait_recv()
                y = y + rs_buf[pl.ds(sz * BLK + HB * h, HB), :]
            inv = lax.rsqrt(jnp.mean(y * y, axis=-1, keepdims=True) + 1e-6)
            out_ref[pl.ds(row_h, HB), :] = y * inv * gamma_ref[...]

            half = out_ref.at[pl.ds(row_h, HB), :]
            for dz in range(1, N_Z):
                tz = (my_z + dz) % N_Z
                rdma = copy(half, half, agz_send.at[dz - 1, h],
                            agz_recv.at[my_z, h], (my_x, my_y, tz))
                rdma.start()
                started.append(rdma)
            send_x(half, agx_recv.at[my_z, h])
            send_y(half, agy_recv.at[my_z, h])

        for h in range(2):
            for dz in range(1, N_Z):
                sz = (my_z + dz) % N_Z
                half = out_ref.at[pl.ds(grp0 + BLK * sz + HB * h, HB), :]
                copy(half, half, agz_send.at[dz - 1, h], agz_recv.at[sz, h],
                     (my_x, my_y, sz)).wait_recv()
                send_x(half, agx_recv.at[sz, h])
                send_y(half, agy_recv.at[sz, h])

        for h in range(2):
            for z in range(N_Z):
                bx = out_ref.at[pl.ds(GRP * p_x + BLK * z + HB * h, HB), :]
                copy(bx, bx, sendx_sems.at[0], agx_recv.at[z, h],
                     xnbr).wait_recv()
                if z % 2 == 1:
                    send_y(bx, agd_recv.at[z, h])
                by = out_ref.at[pl.ds(GRP * p_y + BLK * z + HB * h, HB), :]
                copy(by, by, sendy_sems.at[0], agy_recv.at[z, h],
                     ynbr).wait_recv()
                if z % 2 == 0:
                    send_x(by, agd_recv.at[z, h])

        for h in range(2):
            for z in range(N_Z):
                bd = out_ref.at[pl.ds(GRP * p_dg + BLK * z + HB * h, HB), :]
                src = xnbr if z % 2 == 0 else ynbr
                copy(bd, bd, sendx_sems.at[0], agd_recv.at[z, h],
                     src).wait_recv()

        for rdma in started:
            rdma.wait_send()

    return pl.pallas_call(
        body,
        out_shape=jax.ShapeDtypeStruct((m, d), jnp.float32),
        in_specs=[
            pl.BlockSpec(memory_space=pltpu.VMEM),
            pl.BlockSpec(memory_space=pltpu.VMEM),
            pl.BlockSpec(memory_space=pltpu.VMEM),
        ],
        out_specs=pl.BlockSpec(memory_space=pltpu.VMEM),
        scratch_shapes=[
            pltpu.VMEM((N_Z * BLK, d), jnp.float32),
            pltpu.SemaphoreType.DMA((N_Z, 2)),
            pltpu.SemaphoreType.DMA((N_Z - 1, 2)),
            pltpu.SemaphoreType.DMA((N_Z, 2)),
            pltpu.SemaphoreType.DMA((N_Z - 1, 2)),
            pltpu.SemaphoreType.DMA((N_Z, 2)),
            pltpu.SemaphoreType.DMA((N_Z, 2)),
            pltpu.SemaphoreType.DMA((N_Z, 2)),
            pltpu.SemaphoreType.DMA((12,)),
            pltpu.SemaphoreType.DMA((12,)),
        ],
        compiler_params=pltpu.CompilerParams(collective_id=0),
    )(x2d, resid, gamma2d)
